# baseline (speedup 1.0000x reference)
import functools

import jax
import jax.numpy as jnp
from jax import lax
from jax.experimental import pallas as pl
from jax.experimental.pallas import tpu as pltpu
from jax.experimental.pallas import tpu_sc as plsc

B, N1, N2, C_IN, C_OUT = 4, 8192, 2048, 256, 64
TILE = 1024
NC, NS = 2, 16
NW = NC * NS
QW = (B * N1) // NW
CH = 128


def _conv_bn_relu_kernel(f2_ref, w_ref, gamma_ref, beta_ref, f_ref):
    s = jnp.zeros((1, C_OUT), jnp.float32)
    ss = jnp.zeros((1, C_OUT), jnp.float32)
    for b in range(B):
        g = lax.dot_general(
            f2_ref[b], w_ref[...],
            (((0,), (1,)), ((), ())),
            preferred_element_type=jnp.float32,
        )
        s = s + jnp.sum(g, axis=0, keepdims=True)
        ss = ss + jnp.sum(g * g, axis=0, keepdims=True)
        f_ref[b] = jnp.concatenate(
            [g, jnp.zeros((N2, 128 - C_OUT), jnp.float32)], axis=1)
    cnt = float(B * N2)
    mean = s / cnt
    var = ss / cnt - mean * mean
    scale = gamma_ref[...] * lax.rsqrt(var + 1e-5)
    shift = beta_ref[...] - mean * scale
    pad = jnp.zeros((1, 128 - C_OUT), jnp.float32)
    scale = jnp.concatenate([scale, pad], axis=1)
    shift = jnp.concatenate([shift, pad], axis=1)
    for b in range(B):
        f_ref[b] = jnp.maximum(f_ref[b] * scale + shift, 0.0)


def _top3_kernel(p1_ref, p2_ref, n1_ref, n2_ref, gidx_ref, wgt_ref):
    b = pl.program_id(0)
    dots = lax.dot_general(
        p2_ref[0], p1_ref[0], (((1,), (1,)), ((), ())),
        preferred_element_type=jnp.float32,
    )
    d2_0 = n2_ref[0] + (n1_ref[0] - 2.0 * dots)

    m1 = jnp.min(d2_0, axis=0, keepdims=True)
    c1 = d2_0 <= m1
    e = jnp.where(c1, 1.0, 0.0)
    d2a = jnp.where(c1, jnp.inf, d2_0)
    m2 = jnp.min(d2a, axis=0, keepdims=True)
    c2 = d2a <= m2
    e = jnp.where(c2, 128.0, e)
    d2b = jnp.where(c2, jnp.inf, d2a)
    m3 = jnp.min(d2b, axis=0, keepdims=True)
    e = jnp.where(d2b <= m3, 16384.0, e)
    ms = [m1, m2, m3]

    ihi = lax.broadcasted_iota(jnp.int32, (2, N2), 1) // 128
    ilo = lax.broadcasted_iota(jnp.int32, (2, N2), 1) % 128
    isel = lax.broadcasted_iota(jnp.int32, (2, N2), 0)
    iota2 = jnp.where(isel == 0, ihi, ilo).astype(jnp.float32)
    hl = lax.dot_general(
        iota2, e, (((1,), (0,)), ((), ())),
        preferred_element_type=jnp.float32,
    )
    v3 = jnp.floor(hl * (1.0 / 16384.0))
    rem = hl - v3 * 16384.0
    v2 = jnp.floor(rem * (1.0 / 128.0))
    v1 = rem - v2 * 128.0
    js = [v[0:1, :] * 128.0 + v[1:2, :] for v in (v1, v2, v3)]

    recips = [1.0 / (jnp.maximum(m, 0.0) + 1e-8) for m in ms]
    norm = recips[0] + recips[1] + recips[2]
    for k in range(3):
        jk = jnp.minimum(js[k], float(N2 - 1)).astype(jnp.int32)
        gidx_ref[k:k + 1, :] = jk + b * N2
        wgt_ref[k:k + 1, :] = recips[k] / norm


def _sc_interp_kernel(table_hbm, gidx_hbm, wgt_hbm, out_hbm,
                      idx_v, w_v, rows_v, out_v, gsem0, gsem1, osem):
    wid = lax.axis_index("s") * NC + lax.axis_index("c")
    gsems = [gsem0, gsem1]
    nch = QW // CH

    def stage(ci):
        p = ci % 2
        base = wid * QW + ci * CH
        pltpu.sync_copy(gidx_hbm.at[:, pl.ds(base, CH)], idx_v.at[p])
        pltpu.sync_copy(wgt_hbm.at[:, pl.ds(base, CH)], w_v.at[p])
        return [pltpu.async_copy(table_hbm.at[idx_v.at[p, k]],
                                 rows_v.at[p, k], gsems[p])
                for k in range(3)]

    pending = {0: stage(0)}
    out_cps = {}
    for ci in range(nch):
        p = ci % 2
        if ci + 1 < nch:
            pending[ci + 1] = stage(ci + 1)
        for cp in pending.pop(ci):
            cp.wait()
        if ci >= 1:
            out_cps.pop(ci - 1).wait()

        def body(q16, carry):
            q0 = q16 * 16
            wv = [w_v[p, k, pl.ds(q0, 16)] for k in range(3)]
            for l in range(16):
                q = q0 + l
                for c in range(C_OUT // 16):
                    sl = pl.ds(c * 16, 16)
                    out_v[q, sl] = (wv[0][l] * rows_v[p, 0, q, sl]
                                    + wv[1][l] * rows_v[p, 1, q, sl]
                                    + wv[2][l] * rows_v[p, 2, q, sl])
            return carry

        lax.fori_loop(0, CH // 16, body, 0)
        base = wid * QW + ci * CH
        out_cps[ci] = pltpu.async_copy(
            out_v, out_hbm.at[pl.ds(base, CH)], osem)
    out_cps.pop(nch - 1).wait()


def kernel(p1, p2, f2, W, gamma, beta):
    f = pl.pallas_call(
        _conv_bn_relu_kernel,
        out_shape=jax.ShapeDtypeStruct((B, N2, 128), jnp.float32),
    )(f2, W, gamma.reshape(1, C_OUT), beta.reshape(1, C_OUT))

    n1sq = jnp.sum(p1 * p1, axis=2)[:, None, :]
    n2sq = jnp.sum(p2 * p2, axis=2, keepdims=True)

    gidx, wgt = pl.pallas_call(
        _top3_kernel,
        grid=(B, N1 // TILE),
        in_specs=[
            pl.BlockSpec((1, TILE, 3), lambda b, t: (b, t, 0)),
            pl.BlockSpec((1, N2, 3), lambda b, t: (b, 0, 0)),
            pl.BlockSpec((1, 1, TILE), lambda b, t: (b, 0, t)),
            pl.BlockSpec((1, N2, 1), lambda b, t: (b, 0, 0)),
        ],
        out_specs=[
            pl.BlockSpec((3, TILE), lambda b, t: (0, b * (N1 // TILE) + t)),
            pl.BlockSpec((3, TILE), lambda b, t: (0, b * (N1 // TILE) + t)),
        ],
        out_shape=[
            jax.ShapeDtypeStruct((3, B * N1), jnp.int32),
            jax.ShapeDtypeStruct((3, B * N1), jnp.float32),
        ],
    )(p1, p2, n1sq, n2sq)

    table = f.reshape(B * N2, 128)
    idx_flat = gidx
    w_flat = wgt

    sc_fn = functools.partial(
        pl.kernel,
        out_type=jax.ShapeDtypeStruct((B * N1, C_OUT), jnp.float32),
        mesh=plsc.VectorSubcoreMesh(core_axis_name="c", subcore_axis_name="s"),
        scratch_types=[
            pltpu.VMEM((2, 3, CH), jnp.int32),
            pltpu.VMEM((2, 3, CH), jnp.float32),
            pltpu.VMEM((2, 3, CH, 128), jnp.float32),
            pltpu.VMEM((CH, C_OUT), jnp.float32),
            pltpu.SemaphoreType.DMA,
            pltpu.SemaphoreType.DMA,
            pltpu.SemaphoreType.DMA,
        ],
    )(_sc_interp_kernel)
    out_flat = sc_fn(table, idx_flat, w_flat)

    return out_flat.reshape(B, N1, C_OUT).transpose(0, 2, 1)


# speedup vs baseline: 1.0253x; 1.0253x over previous
import functools

import jax
import jax.numpy as jnp
from jax import lax
from jax.experimental import pallas as pl
from jax.experimental.pallas import tpu as pltpu
from jax.experimental.pallas import tpu_sc as plsc

B, N1, N2, C_IN, C_OUT = 4, 8192, 2048, 256, 64
TILE = 1024
NC, NS = 2, 16
NW = NC * NS
QW = (B * N1) // NW
CH = 128


def _conv_bn_relu_kernel(f2_ref, w_ref, gamma_ref, beta_ref, f_ref):
    s = jnp.zeros((1, C_OUT), jnp.float32)
    ss = jnp.zeros((1, C_OUT), jnp.float32)
    for b in range(B):
        g = lax.dot_general(
            f2_ref[b], w_ref[...],
            (((0,), (1,)), ((), ())),
            preferred_element_type=jnp.float32,
        )
        s = s + jnp.sum(g, axis=0, keepdims=True)
        ss = ss + jnp.sum(g * g, axis=0, keepdims=True)
        f_ref[b] = jnp.concatenate(
            [g, jnp.zeros((N2, 128 - C_OUT), jnp.float32)], axis=1)
    cnt = float(B * N2)
    mean = s / cnt
    var = ss / cnt - mean * mean
    scale = gamma_ref[...] * lax.rsqrt(var + 1e-5)
    shift = beta_ref[...] - mean * scale
    pad = jnp.zeros((1, 128 - C_OUT), jnp.float32)
    scale = jnp.concatenate([scale, pad], axis=1)
    shift = jnp.concatenate([shift, pad], axis=1)
    for b in range(B):
        f_ref[b] = jnp.maximum(f_ref[b] * scale + shift, 0.0)


def _top3_kernel(p1_ref, p2_ref, n1_ref, n2_ref, gidx_ref, wgt_ref):
    b = pl.program_id(0)
    dots = lax.dot_general(
        p2_ref[0], p1_ref[0], (((1,), (1,)), ((), ())),
        preferred_element_type=jnp.float32,
    )
    d2_0 = n2_ref[0] + (n1_ref[0] - 2.0 * dots)

    m1 = jnp.min(d2_0, axis=0, keepdims=True)
    d2a = jnp.where(d2_0 <= m1, jnp.inf, d2_0)
    m2 = jnp.min(d2a, axis=0, keepdims=True)
    d2b = jnp.where(d2a <= m2, jnp.inf, d2a)
    m3 = jnp.min(d2b, axis=0, keepdims=True)
    ms = [m1, m2, m3]

    ihi = lax.broadcasted_iota(jnp.int32, (2, N2), 1) // 128
    ilo = lax.broadcasted_iota(jnp.int32, (2, N2), 1) % 128
    isel = lax.broadcasted_iota(jnp.int32, (2, N2), 0)
    iota2 = jnp.where(isel == 0, ihi, ilo).astype(jnp.float32)
    js = []
    for k, (dk, mk) in enumerate(zip((d2_0, d2a, d2b), ms)):
        eqf = jnp.where(dk == mk, 1.0, 0.0)
        hl = lax.dot_general(
            iota2, eqf, (((1,), (0,)), ((), ())),
            preferred_element_type=jnp.float32,
        )
        js.append(hl[0:1, :] * 128.0 + hl[1:2, :])

    recips = [1.0 / (jnp.maximum(m, 0.0) + 1e-8) for m in ms]
    norm = recips[0] + recips[1] + recips[2]
    for k in range(3):
        jk = jnp.minimum(js[k], float(N2 - 1)).astype(jnp.int32)
        gidx_ref[k:k + 1, :] = jk + b * N2
        wgt_ref[k:k + 1, :] = recips[k] / norm


def _sc_interp_kernel(table_hbm, gidx_hbm, wgt_hbm, out_hbm,
                      idx_v, w_v, rows_v, out_v, gsem0, gsem1, osem):
    wid = lax.axis_index("s") * NC + lax.axis_index("c")
    gsems = [gsem0, gsem1]
    nch = QW // CH

    def stage(ci):
        p = ci % 2
        base = wid * QW + ci * CH
        pltpu.sync_copy(gidx_hbm.at[:, pl.ds(base, CH)], idx_v.at[p])
        pltpu.sync_copy(wgt_hbm.at[:, pl.ds(base, CH)], w_v.at[p])
        return [pltpu.async_copy(table_hbm.at[idx_v.at[p, k]],
                                 rows_v.at[p, k], gsems[p])
                for k in range(3)]

    pending = {0: stage(0)}
    out_cps = {}
    for ci in range(nch):
        p = ci % 2
        if ci + 1 < nch:
            pending[ci + 1] = stage(ci + 1)
        for cp in pending.pop(ci):
            cp.wait()
        if ci >= 1:
            out_cps.pop(ci - 1).wait()

        def body(q16, carry):
            q0 = q16 * 16
            wv = [w_v[p, k, pl.ds(q0, 16)] for k in range(3)]
            for l in range(16):
                q = q0 + l
                for c in range(C_OUT // 16):
                    sl = pl.ds(c * 16, 16)
                    out_v[q, sl] = (wv[0][l] * rows_v[p, 0, q, sl]
                                    + wv[1][l] * rows_v[p, 1, q, sl]
                                    + wv[2][l] * rows_v[p, 2, q, sl])
            return carry

        lax.fori_loop(0, CH // 16, body, 0)
        base = wid * QW + ci * CH
        out_cps[ci] = pltpu.async_copy(
            out_v, out_hbm.at[pl.ds(base, CH)], osem)
    out_cps.pop(nch - 1).wait()


def kernel(p1, p2, f2, W, gamma, beta):
    f = pl.pallas_call(
        _conv_bn_relu_kernel,
        out_shape=jax.ShapeDtypeStruct((B, N2, 128), jnp.float32),
    )(f2, W, gamma.reshape(1, C_OUT), beta.reshape(1, C_OUT))

    n1sq = jnp.sum(p1 * p1, axis=2)[:, None, :]
    n2sq = jnp.sum(p2 * p2, axis=2, keepdims=True)

    gidx, wgt = pl.pallas_call(
        _top3_kernel,
        grid=(B, N1 // TILE),
        in_specs=[
            pl.BlockSpec((1, TILE, 3), lambda b, t: (b, t, 0)),
            pl.BlockSpec((1, N2, 3), lambda b, t: (b, 0, 0)),
            pl.BlockSpec((1, 1, TILE), lambda b, t: (b, 0, t)),
            pl.BlockSpec((1, N2, 1), lambda b, t: (b, 0, 0)),
        ],
        out_specs=[
            pl.BlockSpec((3, TILE), lambda b, t: (0, b * (N1 // TILE) + t)),
            pl.BlockSpec((3, TILE), lambda b, t: (0, b * (N1 // TILE) + t)),
        ],
        out_shape=[
            jax.ShapeDtypeStruct((3, B * N1), jnp.int32),
            jax.ShapeDtypeStruct((3, B * N1), jnp.float32),
        ],
    )(p1, p2, n1sq, n2sq)

    table = f.reshape(B * N2, 128)
    idx_flat = gidx
    w_flat = wgt

    sc_fn = functools.partial(
        pl.kernel,
        out_type=jax.ShapeDtypeStruct((B * N1, C_OUT), jnp.float32),
        mesh=plsc.VectorSubcoreMesh(core_axis_name="c", subcore_axis_name="s"),
        scratch_types=[
            pltpu.VMEM((2, 3, CH), jnp.int32),
            pltpu.VMEM((2, 3, CH), jnp.float32),
            pltpu.VMEM((2, 3, CH, 128), jnp.float32),
            pltpu.VMEM((CH, C_OUT), jnp.float32),
            pltpu.SemaphoreType.DMA,
            pltpu.SemaphoreType.DMA,
            pltpu.SemaphoreType.DMA,
        ],
    )(_sc_interp_kernel)
    out_flat = sc_fn(table, idx_flat, w_flat)

    return out_flat.reshape(B, N1, C_OUT).transpose(0, 2, 1)


# speedup vs baseline: 1.0628x; 1.0366x over previous
import functools

import jax
import jax.numpy as jnp
from jax import lax
from jax.experimental import pallas as pl
from jax.experimental.pallas import tpu as pltpu
from jax.experimental.pallas import tpu_sc as plsc

B, N1, N2, C_IN, C_OUT = 4, 8192, 2048, 256, 64
TILE = 2048
NC, NS = 2, 16
NW = NC * NS
QW = (B * N1) // NW
CH = 128


def _conv_bn_relu_kernel(f2_ref, w_ref, gamma_ref, beta_ref, f_ref):
    s = jnp.zeros((1, C_OUT), jnp.float32)
    ss = jnp.zeros((1, C_OUT), jnp.float32)
    for b in range(B):
        g = lax.dot_general(
            f2_ref[b], w_ref[...],
            (((0,), (1,)), ((), ())),
            preferred_element_type=jnp.float32,
        )
        s = s + jnp.sum(g, axis=0, keepdims=True)
        ss = ss + jnp.sum(g * g, axis=0, keepdims=True)
        f_ref[b] = jnp.concatenate(
            [g, jnp.zeros((N2, 128 - C_OUT), jnp.float32)], axis=1)
    cnt = float(B * N2)
    mean = s / cnt
    var = ss / cnt - mean * mean
    scale = gamma_ref[...] * lax.rsqrt(var + 1e-5)
    shift = beta_ref[...] - mean * scale
    pad = jnp.zeros((1, 128 - C_OUT), jnp.float32)
    scale = jnp.concatenate([scale, pad], axis=1)
    shift = jnp.concatenate([shift, pad], axis=1)
    for b in range(B):
        f_ref[b] = jnp.maximum(f_ref[b] * scale + shift, 0.0)


def _top3_kernel(p1_ref, p2_ref, n1_ref, n2_ref, gidx_ref, wgt_ref):
    b = pl.program_id(0)
    dots = lax.dot_general(
        p2_ref[0], p1_ref[0], (((1,), (1,)), ((), ())),
        preferred_element_type=jnp.float32,
    )
    d2_0 = n2_ref[0] + (n1_ref[0] - 2.0 * dots)

    m1 = jnp.min(d2_0, axis=0, keepdims=True)
    d2a = jnp.where(d2_0 <= m1, jnp.inf, d2_0)
    m2 = jnp.min(d2a, axis=0, keepdims=True)
    d2b = jnp.where(d2a <= m2, jnp.inf, d2a)
    m3 = jnp.min(d2b, axis=0, keepdims=True)
    ms = [m1, m2, m3]

    ihi = lax.broadcasted_iota(jnp.int32, (2, N2), 1) // 128
    ilo = lax.broadcasted_iota(jnp.int32, (2, N2), 1) % 128
    isel = lax.broadcasted_iota(jnp.int32, (2, N2), 0)
    iota2 = jnp.where(isel == 0, ihi, ilo).astype(jnp.float32)
    js = []
    for k, (dk, mk) in enumerate(zip((d2_0, d2a, d2b), ms)):
        eqf = jnp.where(dk == mk, 1.0, 0.0)
        hl = lax.dot_general(
            iota2, eqf, (((1,), (0,)), ((), ())),
            preferred_element_type=jnp.float32,
        )
        js.append(hl[0:1, :] * 128.0 + hl[1:2, :])

    recips = [1.0 / (jnp.maximum(m, 0.0) + 1e-8) for m in ms]
    norm = recips[0] + recips[1] + recips[2]
    for k in range(3):
        jk = jnp.minimum(js[k], float(N2 - 1)).astype(jnp.int32)
        gidx_ref[k:k + 1, :] = jk + b * N2
        wgt_ref[k:k + 1, :] = recips[k] / norm


def _sc_interp_kernel(table_hbm, gidx_hbm, wgt_hbm, out_hbm,
                      idx_v, w_v, rows_v, out_v, gsem0, gsem1, osem):
    wid = lax.axis_index("s") * NC + lax.axis_index("c")
    gsems = [gsem0, gsem1]
    nch = QW // CH

    def stage(ci):
        p = ci % 2
        base = wid * QW + ci * CH
        pltpu.sync_copy(gidx_hbm.at[:, pl.ds(base, CH)], idx_v.at[p])
        pltpu.sync_copy(wgt_hbm.at[:, pl.ds(base, CH)], w_v.at[p])
        return [pltpu.async_copy(table_hbm.at[idx_v.at[p, k]],
                                 rows_v.at[p, k], gsems[p])
                for k in range(3)]

    pending = {0: stage(0)}
    out_cps = {}
    for ci in range(nch):
        p = ci % 2
        if ci + 1 < nch:
            pending[ci + 1] = stage(ci + 1)
        for cp in pending.pop(ci):
            cp.wait()
        if ci >= 1:
            out_cps.pop(ci - 1).wait()

        def body(q16, carry):
            q0 = q16 * 16
            wv = [w_v[p, k, pl.ds(q0, 16)] for k in range(3)]
            for l in range(16):
                q = q0 + l
                for c in range(C_OUT // 16):
                    sl = pl.ds(c * 16, 16)
                    out_v[q, sl] = (wv[0][l] * rows_v[p, 0, q, sl]
                                    + wv[1][l] * rows_v[p, 1, q, sl]
                                    + wv[2][l] * rows_v[p, 2, q, sl])
            return carry

        lax.fori_loop(0, CH // 16, body, 0)
        base = wid * QW + ci * CH
        out_cps[ci] = pltpu.async_copy(
            out_v, out_hbm.at[pl.ds(base, CH)], osem)
    out_cps.pop(nch - 1).wait()


def kernel(p1, p2, f2, W, gamma, beta):
    f = pl.pallas_call(
        _conv_bn_relu_kernel,
        out_shape=jax.ShapeDtypeStruct((B, N2, 128), jnp.float32),
    )(f2, W, gamma.reshape(1, C_OUT), beta.reshape(1, C_OUT))

    n1sq = jnp.sum(p1 * p1, axis=2)[:, None, :]
    n2sq = jnp.sum(p2 * p2, axis=2, keepdims=True)

    gidx, wgt = pl.pallas_call(
        _top3_kernel,
        grid=(B, N1 // TILE),
        in_specs=[
            pl.BlockSpec((1, TILE, 3), lambda b, t: (b, t, 0)),
            pl.BlockSpec((1, N2, 3), lambda b, t: (b, 0, 0)),
            pl.BlockSpec((1, 1, TILE), lambda b, t: (b, 0, t)),
            pl.BlockSpec((1, N2, 1), lambda b, t: (b, 0, 0)),
        ],
        out_specs=[
            pl.BlockSpec((3, TILE), lambda b, t: (0, b * (N1 // TILE) + t)),
            pl.BlockSpec((3, TILE), lambda b, t: (0, b * (N1 // TILE) + t)),
        ],
        out_shape=[
            jax.ShapeDtypeStruct((3, B * N1), jnp.int32),
            jax.ShapeDtypeStruct((3, B * N1), jnp.float32),
        ],
    )(p1, p2, n1sq, n2sq)

    table = f.reshape(B * N2, 128)
    idx_flat = gidx
    w_flat = wgt

    sc_fn = functools.partial(
        pl.kernel,
        out_type=jax.ShapeDtypeStruct((B * N1, C_OUT), jnp.float32),
        mesh=plsc.VectorSubcoreMesh(core_axis_name="c", subcore_axis_name="s"),
        scratch_types=[
            pltpu.VMEM((2, 3, CH), jnp.int32),
            pltpu.VMEM((2, 3, CH), jnp.float32),
            pltpu.VMEM((2, 3, CH, 128), jnp.float32),
            pltpu.VMEM((CH, C_OUT), jnp.float32),
            pltpu.SemaphoreType.DMA,
            pltpu.SemaphoreType.DMA,
            pltpu.SemaphoreType.DMA,
        ],
    )(_sc_interp_kernel)
    out_flat = sc_fn(table, idx_flat, w_flat)

    return out_flat.reshape(B, N1, C_OUT).transpose(0, 2, 1)


# speedup vs baseline: 1.1371x; 1.0700x over previous
import functools

import jax
import jax.numpy as jnp
from jax import lax
from jax.experimental import pallas as pl
from jax.experimental.pallas import tpu as pltpu
from jax.experimental.pallas import tpu_sc as plsc

B, N1, N2, C_IN, C_OUT = 4, 8192, 2048, 256, 64
TILE = 2048
NC, NS = 2, 16
NW = NC * NS
QW = (B * N1) // NW
CH = 128


def _conv_bn_relu_kernel(f2_ref, w_ref, gamma_ref, beta_ref, f_ref):
    s = jnp.zeros((1, C_OUT), jnp.float32)
    ss = jnp.zeros((1, C_OUT), jnp.float32)
    for b in range(B):
        g = lax.dot_general(
            f2_ref[b], w_ref[...],
            (((0,), (1,)), ((), ())),
            preferred_element_type=jnp.float32,
        )
        s = s + jnp.sum(g, axis=0, keepdims=True)
        ss = ss + jnp.sum(g * g, axis=0, keepdims=True)
        f_ref[b] = jnp.concatenate(
            [g, jnp.zeros((N2, 128 - C_OUT), jnp.float32)], axis=1)
    cnt = float(B * N2)
    mean = s / cnt
    var = ss / cnt - mean * mean
    scale = gamma_ref[...] * lax.rsqrt(var + 1e-5)
    shift = beta_ref[...] - mean * scale
    pad = jnp.zeros((1, 128 - C_OUT), jnp.float32)
    scale = jnp.concatenate([scale, pad], axis=1)
    shift = jnp.concatenate([shift, pad], axis=1)
    for b in range(B):
        f_ref[b] = jnp.maximum(f_ref[b] * scale + shift, 0.0)


def _top3_kernel(p1_ref, p2_ref, n1_ref, n2_ref, gidx_ref, wgt_ref):
    b = pl.program_id(0)
    dots = lax.dot_general(
        p2_ref[0], p1_ref[0], (((1,), (1,)), ((), ())),
        preferred_element_type=jnp.float32,
    )
    d2_0 = n2_ref[0] + (n1_ref[0] - 2.0 * dots)

    m1 = jnp.min(d2_0, axis=0, keepdims=True)
    c1 = d2_0 <= m1
    d2a = jnp.where(c1, jnp.inf, d2_0)
    m2 = jnp.min(d2a, axis=0, keepdims=True)
    c2 = d2a <= m2
    d2b = jnp.where(c2, jnp.inf, d2a)
    m3 = jnp.min(d2b, axis=0, keepdims=True)
    c3 = d2b <= m3
    ms = [m1, m2, m3]

    ihi = lax.broadcasted_iota(jnp.int32, (2, N2), 1) // 128
    ilo = lax.broadcasted_iota(jnp.int32, (2, N2), 1) % 128
    isel = lax.broadcasted_iota(jnp.int32, (2, N2), 0)
    iota2 = jnp.where(isel == 0, ihi, ilo).astype(jnp.float32)
    js = []
    for ck in (c1, c2, c3):
        eqf = jnp.where(ck, 1.0, 0.0)
        hl = lax.dot_general(
            iota2, eqf, (((1,), (0,)), ((), ())),
            preferred_element_type=jnp.float32,
        )
        js.append(hl[0:1, :] * 128.0 + hl[1:2, :])

    recips = [1.0 / (jnp.maximum(m, 0.0) + 1e-8) for m in ms]
    norm = recips[0] + recips[1] + recips[2]
    for k in range(3):
        jk = jnp.minimum(js[k], float(N2 - 1)).astype(jnp.int32)
        gidx_ref[k:k + 1, :] = jk + b * N2
        wgt_ref[k:k + 1, :] = recips[k] / norm


def _sc_interp_kernel(table_hbm, gidx_hbm, wgt_hbm, out_hbm,
                      idx_v, w_v, rows_v, out_v, gsem0, gsem1, osem):
    wid = lax.axis_index("s") * NC + lax.axis_index("c")
    gsems = [gsem0, gsem1]
    nch = QW // CH

    def stage(ci):
        p = ci % 2
        base = wid * QW + ci * CH
        pltpu.sync_copy(gidx_hbm.at[:, pl.ds(base, CH)], idx_v.at[p])
        pltpu.sync_copy(wgt_hbm.at[:, pl.ds(base, CH)], w_v.at[p])
        return [pltpu.async_copy(table_hbm.at[idx_v.at[p, k]],
                                 rows_v.at[p, k], gsems[p])
                for k in range(3)]

    pending = {0: stage(0)}
    out_cps = {}
    for ci in range(nch):
        p = ci % 2
        if ci + 1 < nch:
            pending[ci + 1] = stage(ci + 1)
        for cp in pending.pop(ci):
            cp.wait()
        if ci >= 1:
            out_cps.pop(ci - 1).wait()

        def body(q16, carry):
            q0 = q16 * 16
            wv = [w_v[p, k, pl.ds(q0, 16)] for k in range(3)]
            for l in range(16):
                q = q0 + l
                for c in range(C_OUT // 16):
                    sl = pl.ds(c * 16, 16)
                    out_v[q, sl] = (wv[0][l] * rows_v[p, 0, q, sl]
                                    + wv[1][l] * rows_v[p, 1, q, sl]
                                    + wv[2][l] * rows_v[p, 2, q, sl])
            return carry

        lax.fori_loop(0, CH // 16, body, 0)
        base = wid * QW + ci * CH
        out_cps[ci] = pltpu.async_copy(
            out_v, out_hbm.at[pl.ds(base, CH)], osem)
    out_cps.pop(nch - 1).wait()


def kernel(p1, p2, f2, W, gamma, beta):
    f = pl.pallas_call(
        _conv_bn_relu_kernel,
        out_shape=jax.ShapeDtypeStruct((B, N2, 128), jnp.float32),
    )(f2, W, gamma.reshape(1, C_OUT), beta.reshape(1, C_OUT))

    n1sq = jnp.sum(p1 * p1, axis=2)[:, None, :]
    n2sq = jnp.sum(p2 * p2, axis=2, keepdims=True)

    gidx, wgt = pl.pallas_call(
        _top3_kernel,
        grid=(B, N1 // TILE),
        in_specs=[
            pl.BlockSpec((1, TILE, 3), lambda b, t: (b, t, 0)),
            pl.BlockSpec((1, N2, 3), lambda b, t: (b, 0, 0)),
            pl.BlockSpec((1, 1, TILE), lambda b, t: (b, 0, t)),
            pl.BlockSpec((1, N2, 1), lambda b, t: (b, 0, 0)),
        ],
        out_specs=[
            pl.BlockSpec((3, TILE), lambda b, t: (0, b * (N1 // TILE) + t)),
            pl.BlockSpec((3, TILE), lambda b, t: (0, b * (N1 // TILE) + t)),
        ],
        out_shape=[
            jax.ShapeDtypeStruct((3, B * N1), jnp.int32),
            jax.ShapeDtypeStruct((3, B * N1), jnp.float32),
        ],
    )(p1, p2, n1sq, n2sq)

    table = f.reshape(B * N2, 128)
    idx_flat = gidx
    w_flat = wgt

    sc_fn = functools.partial(
        pl.kernel,
        out_type=jax.ShapeDtypeStruct((B * N1, C_OUT), jnp.float32),
        mesh=plsc.VectorSubcoreMesh(core_axis_name="c", subcore_axis_name="s"),
        scratch_types=[
            pltpu.VMEM((2, 3, CH), jnp.int32),
            pltpu.VMEM((2, 3, CH), jnp.float32),
            pltpu.VMEM((2, 3, CH, 128), jnp.float32),
            pltpu.VMEM((CH, C_OUT), jnp.float32),
            pltpu.SemaphoreType.DMA,
            pltpu.SemaphoreType.DMA,
            pltpu.SemaphoreType.DMA,
        ],
    )(_sc_interp_kernel)
    out_flat = sc_fn(table, idx_flat, w_flat)

    return out_flat.reshape(B, N1, C_OUT).transpose(0, 2, 1)


# speedup vs baseline: 1.1622x; 1.0221x over previous
import functools

import jax
import jax.numpy as jnp
from jax import lax
from jax.experimental import pallas as pl
from jax.experimental.pallas import tpu as pltpu
from jax.experimental.pallas import tpu_sc as plsc

B, N1, N2, C_IN, C_OUT = 4, 8192, 2048, 256, 64
TILE = 2048
NC, NS = 2, 16
NW = NC * NS
QW = (B * N1) // NW
CH = 128


def _conv_bn_relu_kernel(f2_ref, w_ref, gamma_ref, beta_ref, f_ref):
    s = jnp.zeros((1, C_OUT), jnp.float32)
    ss = jnp.zeros((1, C_OUT), jnp.float32)
    for b in range(B):
        g = lax.dot_general(
            f2_ref[b], w_ref[...],
            (((0,), (1,)), ((), ())),
            preferred_element_type=jnp.float32,
        )
        s = s + jnp.sum(g, axis=0, keepdims=True)
        ss = ss + jnp.sum(g * g, axis=0, keepdims=True)
        f_ref[b] = jnp.concatenate(
            [g, jnp.zeros((N2, 128 - C_OUT), jnp.float32)], axis=1)
    cnt = float(B * N2)
    mean = s / cnt
    var = ss / cnt - mean * mean
    scale = gamma_ref[...] * lax.rsqrt(var + 1e-5)
    shift = beta_ref[...] - mean * scale
    pad = jnp.zeros((1, 128 - C_OUT), jnp.float32)
    scale = jnp.concatenate([scale, pad], axis=1)
    shift = jnp.concatenate([shift, pad], axis=1)
    for b in range(B):
        f_ref[b] = jnp.maximum(f_ref[b] * scale + shift, 0.0)


def _top3_kernel(p1_ref, p2_ref, n1_ref, n2_ref, gidx_ref, wgt_ref):
    b = pl.program_id(0)
    dots = lax.dot_general(
        p2_ref[0], p1_ref[0], (((1,), (1,)), ((), ())),
        preferred_element_type=jnp.float32,
    )
    d2_0 = n2_ref[0] - 2.0 * dots

    m1 = jnp.min(d2_0, axis=0, keepdims=True)
    c1 = d2_0 <= m1
    d2a = jnp.where(c1, jnp.inf, d2_0)
    m2 = jnp.min(d2a, axis=0, keepdims=True)
    c2 = d2a <= m2
    d2b = jnp.where(c2, jnp.inf, d2a)
    m3 = jnp.min(d2b, axis=0, keepdims=True)
    c3 = d2b <= m3
    ms = [m + n1_ref[0] for m in (m1, m2, m3)]

    ihi = lax.broadcasted_iota(jnp.int32, (2, N2), 1) // 128
    ilo = lax.broadcasted_iota(jnp.int32, (2, N2), 1) % 128
    isel = lax.broadcasted_iota(jnp.int32, (2, N2), 0)
    iota2 = jnp.where(isel == 0, ihi, ilo).astype(jnp.float32)
    js = []
    for ck in (c1, c2, c3):
        eqf = jnp.where(ck, 1.0, 0.0)
        hl = lax.dot_general(
            iota2, eqf, (((1,), (0,)), ((), ())),
            preferred_element_type=jnp.float32,
        )
        js.append(hl[0:1, :] * 128.0 + hl[1:2, :])

    recips = [1.0 / (jnp.maximum(m, 0.0) + 1e-8) for m in ms]
    norm = recips[0] + recips[1] + recips[2]
    for k in range(3):
        jk = jnp.minimum(js[k], float(N2 - 1)).astype(jnp.int32)
        gidx_ref[k:k + 1, :] = jk + b * N2
        wgt_ref[k:k + 1, :] = recips[k] / norm


def _sc_interp_kernel(table_hbm, gidx_hbm, wgt_hbm, out_hbm,
                      idx_v, w_v, rows_v, out_v, gsem0, gsem1, osem):
    wid = lax.axis_index("s") * NC + lax.axis_index("c")
    gsems = [gsem0, gsem1]
    nch = QW // CH

    def stage(ci):
        p = ci % 2
        base = wid * QW + ci * CH
        pltpu.sync_copy(gidx_hbm.at[:, pl.ds(base, CH)], idx_v.at[p])
        pltpu.sync_copy(wgt_hbm.at[:, pl.ds(base, CH)], w_v.at[p])
        return [pltpu.async_copy(table_hbm.at[idx_v.at[p, k]],
                                 rows_v.at[p, k], gsems[p])
                for k in range(3)]

    pending = {0: stage(0)}
    out_cps = {}
    for ci in range(nch):
        p = ci % 2
        if ci + 1 < nch:
            pending[ci + 1] = stage(ci + 1)
        for cp in pending.pop(ci):
            cp.wait()
        if ci >= 1:
            out_cps.pop(ci - 1).wait()

        def body(q16, carry):
            q0 = q16 * 16
            wv = [w_v[p, k, pl.ds(q0, 16)] for k in range(3)]
            for l in range(16):
                q = q0 + l
                for c in range(C_OUT // 16):
                    sl = pl.ds(c * 16, 16)
                    out_v[q, sl] = (wv[0][l] * rows_v[p, 0, q, sl]
                                    + wv[1][l] * rows_v[p, 1, q, sl]
                                    + wv[2][l] * rows_v[p, 2, q, sl])
            return carry

        lax.fori_loop(0, CH // 16, body, 0)
        base = wid * QW + ci * CH
        out_cps[ci] = pltpu.async_copy(
            out_v, out_hbm.at[pl.ds(base, CH)], osem)
    out_cps.pop(nch - 1).wait()


def kernel(p1, p2, f2, W, gamma, beta):
    f = pl.pallas_call(
        _conv_bn_relu_kernel,
        out_shape=jax.ShapeDtypeStruct((B, N2, 128), jnp.float32),
    )(f2, W, gamma.reshape(1, C_OUT), beta.reshape(1, C_OUT))

    n1sq = jnp.sum(p1 * p1, axis=2)[:, None, :]
    n2sq = jnp.sum(p2 * p2, axis=2, keepdims=True)

    gidx, wgt = pl.pallas_call(
        _top3_kernel,
        grid=(B, N1 // TILE),
        in_specs=[
            pl.BlockSpec((1, TILE, 3), lambda b, t: (b, t, 0)),
            pl.BlockSpec((1, N2, 3), lambda b, t: (b, 0, 0)),
            pl.BlockSpec((1, 1, TILE), lambda b, t: (b, 0, t)),
            pl.BlockSpec((1, N2, 1), lambda b, t: (b, 0, 0)),
        ],
        out_specs=[
            pl.BlockSpec((3, TILE), lambda b, t: (0, b * (N1 // TILE) + t)),
            pl.BlockSpec((3, TILE), lambda b, t: (0, b * (N1 // TILE) + t)),
        ],
        out_shape=[
            jax.ShapeDtypeStruct((3, B * N1), jnp.int32),
            jax.ShapeDtypeStruct((3, B * N1), jnp.float32),
        ],
    )(p1, p2, n1sq, n2sq)

    table = f.reshape(B * N2, 128)
    idx_flat = gidx
    w_flat = wgt

    sc_fn = functools.partial(
        pl.kernel,
        out_type=jax.ShapeDtypeStruct((B * N1, C_OUT), jnp.float32),
        mesh=plsc.VectorSubcoreMesh(core_axis_name="c", subcore_axis_name="s"),
        scratch_types=[
            pltpu.VMEM((2, 3, CH), jnp.int32),
            pltpu.VMEM((2, 3, CH), jnp.float32),
            pltpu.VMEM((2, 3, CH, 128), jnp.float32),
            pltpu.VMEM((CH, C_OUT), jnp.float32),
            pltpu.SemaphoreType.DMA,
            pltpu.SemaphoreType.DMA,
            pltpu.SemaphoreType.DMA,
        ],
    )(_sc_interp_kernel)
    out_flat = sc_fn(table, idx_flat, w_flat)

    return out_flat.reshape(B, N1, C_OUT).transpose(0, 2, 1)
